# baseline (speedup 1.0000x reference)
import functools
import jax
import jax.numpy as jnp
from jax import lax
from jax.experimental import pallas as pl
from jax.experimental.pallas import tpu as pltpu
from jax.experimental.pallas import tpu_sc as plsc

N = 10000
FIN = 128
HID = 64
HEADS = 8
G = 64

NP = 10240
DUMMY = N

NC, NS, L = 2, 16, 16
NW = NC * NS
K = 128
E_TOT = 320000 + N
S0, S1 = 68, 96
SMX = max(S0, S1)
ET_PAD = NS * K * (S0 + S1)
PACK = 16384
RB = NP // 256
STR = NP // NS
W = 2 * HID

_HI = jax.lax.Precision.HIGHEST


def _dot(a, b):
    return jax.lax.dot_general(a, b, (((1,), (0,)), ((), ())),
                               precision=_HI, preferred_element_type=jnp.float32)


def _k1a_body(x_ref, w_ref, h_ref):
    h_ref[0] = _dot(x_ref[...], w_ref[0])


def _mm_heads(xp, w1r):
    return pl.pallas_call(
        _k1a_body,
        grid=(HEADS // 2, RB),
        in_specs=[
            pl.BlockSpec((256, FIN), lambda p, i: (i, 0)),
            pl.BlockSpec((1, FIN, W), lambda p, i: (p, 0, 0)),
        ],
        out_specs=pl.BlockSpec((1, 256, W), lambda p, i: (p, i, 0)),
        out_shape=jax.ShapeDtypeStruct((HEADS // 2, NP, W), jnp.float32),
    )(xp, w1r)


def _k1b_body(x_ref, w_ref, o_ref):
    o_ref[...] = _dot(x_ref[...], w_ref[...])


def _mm_asad(xp, w1sd):
    return pl.pallas_call(
        _k1b_body,
        grid=(RB,),
        in_specs=[
            pl.BlockSpec((256, FIN), lambda i: (i, 0)),
            pl.BlockSpec((FIN, 2 * HEADS), lambda i: (0, 0)),
        ],
        out_specs=pl.BlockSpec((256, 2 * HEADS), lambda i: (i, 0)),
        out_shape=jax.ShapeDtypeStruct((NP, 2 * HEADS), jnp.float32),
    )(xp, w1sd)


def _sc_pass(hh, as0, ad0, as1, ad1, dump, idx_v, eb_v, sc_v, al_v, rows_v,
             out_sh, d0_sh, d1_sh, sems, zb, zs, stripe, nh):
    pltpu.sync_copy(zb.at[stripe], out_sh.at[stripe])
    pltpu.sync_copy(zs.at[stripe], d0_sh.at[stripe])
    pltpu.sync_copy(zs.at[stripe], d1_sh.at[stripe])

    plsc.subcore_barrier()

    def issue(t, b):
        for g in range(K // L):
            gs = pl.ds(g * L, L)
            pk = idx_v[t, gs]
            eb_v[2 * b, gs] = pk & (PACK - 1)
            eb_v[2 * b + 1, gs] = lax.shift_right_logical(pk, 14)
        src_r = eb_v.at[2 * b]
        dst_r = eb_v.at[2 * b + 1]
        pltpu.async_copy(as0.at[src_r], sc_v.at[4 * b + 0], sems[b])
        pltpu.async_copy(ad0.at[dst_r], sc_v.at[4 * b + 1], sems[b])
        pltpu.async_copy(as1.at[src_r], sc_v.at[4 * b + 2], sems[b])
        pltpu.async_copy(ad1.at[dst_r], sc_v.at[4 * b + 3], sems[b])
        pltpu.async_copy(hh.at[src_r], rows_v.at[b], sems[b])

    def wait(b):
        pltpu.make_async_copy(as0.at[pl.ds(0, K)], sc_v.at[4 * b + 0], sems[b]).wait()
        pltpu.make_async_copy(ad0.at[pl.ds(0, K)], sc_v.at[4 * b + 1], sems[b]).wait()
        pltpu.make_async_copy(as1.at[pl.ds(0, K)], sc_v.at[4 * b + 2], sems[b]).wait()
        pltpu.make_async_copy(ad1.at[pl.ds(0, K)], sc_v.at[4 * b + 3], sems[b]).wait()
        pltpu.make_async_copy(hh.at[pl.ds(0, K)], rows_v.at[b], sems[b]).wait()

    def process(t, b):
        for g in range(K // L):
            gs = pl.ds(g * L, L)
            z0 = sc_v[4 * b + 0, gs] + sc_v[4 * b + 1, gs]
            z0 = jnp.where(z0 >= 0.0, z0, 0.2 * z0)
            al_v[2 * b, gs] = jnp.exp(z0)
            z1 = sc_v[4 * b + 2, gs] + sc_v[4 * b + 3, gs]
            z1 = jnp.where(z1 >= 0.0, z1, 0.2 * z1)
            al_v[2 * b + 1, gs] = jnp.exp(z1)
        def scale(e, c2):
            av0 = plsc.load_gather(al_v.at[2 * b], [jnp.full((L,), e, jnp.int32)])
            av1 = plsc.load_gather(al_v.at[2 * b + 1], [jnp.full((L,), e, jnp.int32)])
            for c in range(HID // L):
                rows_v[b, e, pl.ds(c * L, L)] = \
                    rows_v[b, e, pl.ds(c * L, L)] * av0
                rows_v[b, e, pl.ds(HID + c * L, L)] = \
                    rows_v[b, e, pl.ds(HID + c * L, L)] * av1
            return c2
        lax.fori_loop(0, K, scale, 0, unroll=4)
        for g in range(K // L):
            gs = pl.ds(g * L, L)
            eb_v[4, gs] = lax.shift_right_logical(idx_v[t, gs], 14)
        dsc = eb_v.at[4]
        pltpu.sync_copy(rows_v.at[b], out_sh.at[dsc], add=True)
        pltpu.sync_copy(al_v.at[2 * b], d0_sh.at[dsc], add=True)
        pltpu.sync_copy(al_v.at[2 * b + 1], d1_sh.at[dsc], add=True)

    issue(0, 0)
    issue(1, 1)

    def pair(tt, carry):
        t0 = 2 * tt
        wait(0)
        process(t0, 0)

        @pl.when(tt < nh - 1)
        def _():
            issue(t0 + 2, 0)

        wait(1)
        process(t0 + 1, 1)

        @pl.when(tt < nh - 1)
        def _():
            issue(t0 + 3, 1)

        return carry

    lax.fori_loop(0, nh, pair, 0)

    plsc.subcore_barrier()
    dump()
    plsc.subcore_barrier()


def _sc_body(h0, h1, h2, h3,
             t0, t1, t2, t3, t4, t5, t6, t7,
             t8, t9, t10, t11, t12, t13, t14, t15,
             epk, zb, zs, out, den,
             idx_v, eb_v, sc_v, al_v, rows_v,
             out_sh, d0_sh, d1_sh, semA, semB):
    cid = lax.axis_index("c")
    sid = lax.axis_index("s")
    wid = sid * NC + cid
    stripe = pl.ds(sid * STR, STR)
    sems = (semA, semB)
    hs = (h0, h1, h2, h3)
    ts = (t0, t1, t2, t3, t4, t5, t6, t7,
          t8, t9, t10, t11, t12, t13, t14, t15)
    nh = jnp.where(cid == 0, S0 // 2, S1 // 2)

    pltpu.sync_copy(epk.at[wid], idx_v)

    for p in range(HEADS // 2):
        def dump(p=p):
            pltpu.sync_copy(out_sh.at[stripe], out.at[p, cid, stripe])
            pltpu.sync_copy(d0_sh.at[stripe], den.at[p, cid, 0, stripe])
            pltpu.sync_copy(d1_sh.at[stripe], den.at[p, cid, 1, stripe])
        _sc_pass(hs[p], ts[2 * p], ts[HEADS + 2 * p],
                 ts[2 * p + 1], ts[HEADS + 2 * p + 1], dump,
                 idx_v, eb_v, sc_v, al_v, rows_v,
                 out_sh, d0_sh, d1_sh, sems, zb, zs, stripe, nh)


def _sc_body2(hh, as0, ad0, as1, ad1, epk, zb, zs, out, den,
              idx_v, eb_v, sc_v, al_v, rows_v,
              out_sh, d0_sh, d1_sh, semA, semB):
    cid = lax.axis_index("c")
    sid = lax.axis_index("s")
    wid = sid * NC + cid
    stripe = pl.ds(sid * STR, STR)

    pltpu.sync_copy(epk.at[wid], idx_v)
    nh = jnp.where(cid == 0, S0 // 2, S1 // 2)

    def dump():
        pltpu.sync_copy(out_sh.at[stripe], out.at[cid, stripe])
        pltpu.sync_copy(d0_sh.at[stripe], den.at[cid, 0, stripe])
        pltpu.sync_copy(d1_sh.at[stripe], den.at[cid, 1, stripe])

    _sc_pass(hh, as0, ad0, as1, ad1, dump,
             idx_v, eb_v, sc_v, al_v, rows_v,
             out_sh, d0_sh, d1_sh, (semA, semB), zb, zs, stripe, nh)


_SC_SCRATCH = [
    pltpu.VMEM((SMX, K), jnp.int32),
    pltpu.VMEM((8, K), jnp.int32),
    pltpu.VMEM((8, K), jnp.float32),
    pltpu.VMEM((4, K), jnp.float32),
    pltpu.VMEM((2, K, W), jnp.float32),
    pltpu.VMEM_SHARED((NP, W), jnp.float32),
    pltpu.VMEM_SHARED((NP,), jnp.float32),
    pltpu.VMEM_SHARED((NP,), jnp.float32),
    pltpu.SemaphoreType.DMA,
    pltpu.SemaphoreType.DMA,
]

_sc_gat1 = functools.partial(
    pl.kernel,
    out_type=(jax.ShapeDtypeStruct((HEADS // 2, NC, NP, W), jnp.float32),
              jax.ShapeDtypeStruct((HEADS // 2, NC, 2, NP), jnp.float32)),
    mesh=plsc.VectorSubcoreMesh(core_axis_name="c", subcore_axis_name="s",
                                num_cores=NC, num_subcores=NS),
    compiler_params=pltpu.CompilerParams(needs_layout_passes=False),
    scratch_types=_SC_SCRATCH,
)(_sc_body)

_sc_gat2 = functools.partial(
    pl.kernel,
    out_type=(jax.ShapeDtypeStruct((NC, NP, W), jnp.float32),
              jax.ShapeDtypeStruct((NC, 2, NP), jnp.float32)),
    mesh=plsc.VectorSubcoreMesh(core_axis_name="c", subcore_axis_name="s",
                                num_cores=NC, num_subcores=NS),
    compiler_params=pltpu.CompilerParams(needs_layout_passes=False),
    scratch_types=_SC_SCRATCH,
)(_sc_body2)


def _k3_body(o_ref, d_ref, b1_ref, w_ref, out_ref):
    i = pl.program_id(0)
    acc = jnp.zeros((256, 128), jnp.float32)
    for h in range(HEADS):
        p, q = h // 2, h % 2
        v = (o_ref[p, 0, :, q * HID:(q + 1) * HID]
             + o_ref[p, 1, :, q * HID:(q + 1) * HID])
        dh = d_ref[p, :, q, pl.ds(i * 256, 256)]
        dd = dh[0] + dh[1] + 1e-16
        v = v / dd[:, None] + b1_ref[h]
        v = jnp.where(v > 0.0, v, jnp.exp(v) - 1.0)
        acc = acc + _dot(v, w_ref[h])
    out_ref[...] = acc


def _combine_l2(out1, den1, b1r, w2cat):
    NPAIR = HEADS // 2
    return pl.pallas_call(
        _k3_body,
        grid=(RB,),
        in_specs=[
            pl.BlockSpec((NPAIR, NC, 256, W), lambda i: (0, 0, i, 0)),
            pl.BlockSpec((NPAIR, NC, 2, NP), lambda i: (0, 0, 0, 0)),
            pl.BlockSpec((HEADS, HID), lambda i: (0, 0)),
            pl.BlockSpec((HEADS, HID, 128), lambda i: (0, 0, 0)),
        ],
        out_specs=pl.BlockSpec((256, 128), lambda i: (i, 0)),
        out_shape=jax.ShapeDtypeStruct((NP, 128), jnp.float32),
    )(out1, den1, b1r, w2cat)


def _k4_body(o_ref, d_ref, b_ref, b2_ref, fcw_ref, fcb_ref, out_ref, sums, counts):
    i = pl.program_id(0)
    p = o_ref[0, :, :HID] + o_ref[1, :, :HID]
    dh = d_ref[:, 0, pl.ds(i * 256, 256)]
    dd = dh[0] + dh[1] + 1e-16
    v = p / dd[:, None] + b2_ref[0]
    v = jnp.where(v > 0.0, v, jnp.exp(v) - 1.0)
    bb = b_ref[0, 0]
    oh = (bb[:, None] == lax.broadcasted_iota(jnp.int32, (256, G), 1)).astype(jnp.float32)
    ps = jax.lax.dot_general(oh, v, (((0,), (0,)), ((), ())),
                             precision=_HI, preferred_element_type=jnp.float32)
    pc = jnp.sum(oh, axis=0)

    @pl.when(i == 0)
    def _():
        sums[...] = jnp.zeros_like(sums)
        counts[...] = jnp.zeros_like(counts)

    sums[...] += ps
    counts[...] += pc[None, :]

    @pl.when(i == RB - 1)
    def _():
        c = jnp.maximum(counts[0, :], 1.0)
        pooled = sums[...] / c[:, None]
        out_ref[...] = _dot(pooled, fcw_ref[...]) + fcb_ref[0]


def _pool_fc(out2, den2, batchr, b2r, fcw, fcb):
    return pl.pallas_call(
        _k4_body,
        grid=(RB,),
        in_specs=[
            pl.BlockSpec((NC, 256, W), lambda i: (0, i, 0)),
            pl.BlockSpec((NC, 2, NP), lambda i: (0, 0, 0)),
            pl.BlockSpec((1, 1, 256), lambda i: (i, 0, 0)),
            pl.BlockSpec((1, HID), lambda i: (0, 0)),
            pl.BlockSpec((HID, 128), lambda i: (0, 0)),
            pl.BlockSpec((1, 128), lambda i: (0, 0)),
        ],
        out_specs=pl.BlockSpec((G, 128), lambda i: (0, 0)),
        out_shape=jax.ShapeDtypeStruct((G, 128), jnp.float32),
        scratch_shapes=[pltpu.VMEM((G, HID), jnp.float32),
                        pltpu.VMEM((1, G), jnp.float32)],
    )(out2, den2, batchr, b2r, fcw, fcb)


def kernel(x, edge_index, batch, W1, att_src1, att_dst1, b1,
           W2, att_src2, att_dst2, b2, fc_w, fc_b):
    f32 = jnp.float32
    w1r = W1.reshape(FIN, HEADS // 2, W).transpose(1, 0, 2)
    w1s = jnp.einsum("fhc,hc->fh", W1.reshape(FIN, HEADS, HID), att_src1)
    w1d = jnp.einsum("fhc,hc->fh", W1.reshape(FIN, HEADS, HID), att_dst1)
    w1sd = jnp.concatenate([w1s, w1d], axis=1)
    w2r = W2.reshape(HEADS, HID, HID)
    w2s = (W2 @ att_src2[0]).reshape(HEADS, HID, 1)
    w2d = (W2 @ att_dst2[0]).reshape(HEADS, HID, 1)
    w2cat = jnp.concatenate(
        [w2r, w2s, w2d, jnp.zeros((HEADS, HID, 128 - HID - 2), f32)], axis=2)
    b1r = b1.reshape(HEADS, HID)
    b2r = b2.reshape(1, HID)
    fcw = jnp.zeros((HID, 128), f32).at[:, :2].set(fc_w)
    fcb = jnp.zeros((1, 128), f32).at[:, :2].set(fc_b)

    xp = jnp.pad(x, ((0, NP - N), (0, 0)))
    loop = jnp.arange(N, dtype=jnp.int32)
    pad = jnp.full((ET_PAD - E_TOT,), DUMMY, jnp.int32)
    srcs = jnp.concatenate([edge_index[0], loop, pad])
    dsts = jnp.concatenate([edge_index[1], loop, pad])
    flat = dsts * PACK + srcs
    dfill = jnp.full((SMX * K,), DUMMY * PACK + DUMMY, jnp.int32)
    rows_list, off = [], 0
    for w in range(NW):
        lw = (S0 if w % NC == 0 else S1) * K
        seg = flat[off:off + lw]
        off += lw
        rows_list.append(jnp.concatenate([seg, dfill[:SMX * K - lw]]))
    epk = jnp.stack(rows_list).reshape(NW, SMX, K)
    batchr = jnp.concatenate(
        [batch, jnp.full((NP - N,), G, jnp.int32)]).reshape(RB, 1, 256)
    zb = jnp.zeros((NP, W), f32)
    zs = jnp.zeros((NP,), f32)

    h4 = _mm_heads(xp, w1r)
    asad = _mm_asad(xp, w1sd)
    asadt = asad.T

    hs = [h4[p] for p in range(HEADS // 2)]
    ts = [asadt[i] for i in range(2 * HEADS)]
    out1, den1 = _sc_gat1(*hs, *ts, epk, zb, zs)

    o2pre = _combine_l2(out1, den1, b1r, w2cat)
    as2 = o2pre[:, HID]
    ad2 = o2pre[:, HID + 1]

    out2, den2 = _sc_gat2(o2pre, as2, ad2, zs, zs, epk, zb, zs)

    logits = _pool_fc(out2, den2, batchr, b2r, fcw, fcb)
    return logits[:, :2]


# speedup vs baseline: 1.1194x; 1.1194x over previous
import functools
import jax
import jax.numpy as jnp
from jax import lax
from jax.experimental import pallas as pl
from jax.experimental.pallas import tpu as pltpu
from jax.experimental.pallas import tpu_sc as plsc

N = 10000
FIN = 128
HID = 64
HEADS = 8
G = 64

NP = 10240
DUMMY = N

NC, NS, L = 2, 16, 16
NW = NC * NS
K = 128
E_TOT = 320000 + N
S0, S1 = 96, 68
SMX = max(S0, S1)
ET_PAD = NS * K * (S0 + S1)
PACK = 16384
RB = NP // 256
STR = NP // NS
W = 2 * HID

_HI = jax.lax.Precision.HIGHEST


def _dot(a, b):
    return jax.lax.dot_general(a, b, (((1,), (0,)), ((), ())),
                               precision=_HI, preferred_element_type=jnp.float32)


def _k1a_body(x_ref, w_ref, h_ref):
    h_ref[0] = _dot(x_ref[...], w_ref[0])


def _mm_heads(xp, w1r):
    return pl.pallas_call(
        _k1a_body,
        grid=(HEADS // 2, RB),
        in_specs=[
            pl.BlockSpec((256, FIN), lambda p, i: (i, 0)),
            pl.BlockSpec((1, FIN, W), lambda p, i: (p, 0, 0)),
        ],
        out_specs=pl.BlockSpec((1, 256, W), lambda p, i: (p, i, 0)),
        out_shape=jax.ShapeDtypeStruct((HEADS // 2, NP, W), jnp.float32),
    )(xp, w1r)


def _k1b_body(x_ref, w_ref, o_ref):
    o_ref[...] = _dot(x_ref[...], w_ref[...])


def _mm_asad(xp, w1sd):
    return pl.pallas_call(
        _k1b_body,
        grid=(RB,),
        in_specs=[
            pl.BlockSpec((256, FIN), lambda i: (i, 0)),
            pl.BlockSpec((FIN, 2 * HEADS), lambda i: (0, 0)),
        ],
        out_specs=pl.BlockSpec((256, 2 * HEADS), lambda i: (i, 0)),
        out_shape=jax.ShapeDtypeStruct((NP, 2 * HEADS), jnp.float32),
    )(xp, w1sd)


def _sc_pass(hh, as0, ad0, as1, ad1, dump, idx_v, eb_v, sc_v, al_v, rows_v,
             out_sh, d0_sh, d1_sh, sems, zb, zs, stripe, nh):
    pltpu.sync_copy(zb.at[stripe], out_sh.at[stripe])
    pltpu.sync_copy(zs.at[stripe], d0_sh.at[stripe])
    pltpu.sync_copy(zs.at[stripe], d1_sh.at[stripe])

    plsc.subcore_barrier()

    def issue(t, b):
        for g in range(K // L):
            gs = pl.ds(g * L, L)
            pk = idx_v[t, gs]
            eb_v[2 * b, gs] = pk & (PACK - 1)
            eb_v[2 * b + 1, gs] = lax.shift_right_logical(pk, 14)
        src_r = eb_v.at[2 * b]
        dst_r = eb_v.at[2 * b + 1]
        pltpu.async_copy(as0.at[src_r], sc_v.at[4 * b + 0], sems[b])
        pltpu.async_copy(ad0.at[dst_r], sc_v.at[4 * b + 1], sems[b])
        pltpu.async_copy(as1.at[src_r], sc_v.at[4 * b + 2], sems[b])
        pltpu.async_copy(ad1.at[dst_r], sc_v.at[4 * b + 3], sems[b])
        pltpu.async_copy(hh.at[src_r], rows_v.at[b], sems[b])

    def wait(b):
        pltpu.make_async_copy(as0.at[pl.ds(0, K)], sc_v.at[4 * b + 0], sems[b]).wait()
        pltpu.make_async_copy(ad0.at[pl.ds(0, K)], sc_v.at[4 * b + 1], sems[b]).wait()
        pltpu.make_async_copy(as1.at[pl.ds(0, K)], sc_v.at[4 * b + 2], sems[b]).wait()
        pltpu.make_async_copy(ad1.at[pl.ds(0, K)], sc_v.at[4 * b + 3], sems[b]).wait()
        pltpu.make_async_copy(hh.at[pl.ds(0, K)], rows_v.at[b], sems[b]).wait()

    def process(t, b):
        for g in range(K // L):
            gs = pl.ds(g * L, L)
            z0 = sc_v[4 * b + 0, gs] + sc_v[4 * b + 1, gs]
            z0 = jnp.where(z0 >= 0.0, z0, 0.2 * z0)
            al_v[2 * b, gs] = jnp.exp(z0)
            z1 = sc_v[4 * b + 2, gs] + sc_v[4 * b + 3, gs]
            z1 = jnp.where(z1 >= 0.0, z1, 0.2 * z1)
            al_v[2 * b + 1, gs] = jnp.exp(z1)
        def scale(e, c2):
            av0 = plsc.load_gather(al_v.at[2 * b], [jnp.full((L,), e, jnp.int32)])
            av1 = plsc.load_gather(al_v.at[2 * b + 1], [jnp.full((L,), e, jnp.int32)])
            for c in range(HID // L):
                rows_v[b, e, pl.ds(c * L, L)] = \
                    rows_v[b, e, pl.ds(c * L, L)] * av0
                rows_v[b, e, pl.ds(HID + c * L, L)] = \
                    rows_v[b, e, pl.ds(HID + c * L, L)] * av1
            return c2
        lax.fori_loop(0, K, scale, 0, unroll=4)
        for g in range(K // L):
            gs = pl.ds(g * L, L)
            eb_v[4, gs] = lax.shift_right_logical(idx_v[t, gs], 14)
        dsc = eb_v.at[4]
        pltpu.sync_copy(rows_v.at[b], out_sh.at[dsc], add=True)
        pltpu.sync_copy(al_v.at[2 * b], d0_sh.at[dsc], add=True)
        pltpu.sync_copy(al_v.at[2 * b + 1], d1_sh.at[dsc], add=True)

    issue(0, 0)
    issue(1, 1)

    def pair(tt, carry):
        t0 = 2 * tt
        wait(0)
        process(t0, 0)

        @pl.when(tt < nh - 1)
        def _():
            issue(t0 + 2, 0)

        wait(1)
        process(t0 + 1, 1)

        @pl.when(tt < nh - 1)
        def _():
            issue(t0 + 3, 1)

        return carry

    lax.fori_loop(0, nh, pair, 0)

    plsc.subcore_barrier()
    dump()
    plsc.subcore_barrier()


def _sc_body(h0, h1, h2, h3,
             t0, t1, t2, t3, t4, t5, t6, t7,
             t8, t9, t10, t11, t12, t13, t14, t15,
             epk, zb, zs, out, den,
             idx_v, eb_v, sc_v, al_v, rows_v,
             out_sh, d0_sh, d1_sh, semA, semB):
    cid = lax.axis_index("c")
    sid = lax.axis_index("s")
    wid = sid * NC + cid
    stripe = pl.ds(sid * STR, STR)
    sems = (semA, semB)
    hs = (h0, h1, h2, h3)
    ts = (t0, t1, t2, t3, t4, t5, t6, t7,
          t8, t9, t10, t11, t12, t13, t14, t15)
    nh = jnp.where(cid == 0, S0 // 2, S1 // 2)

    pltpu.sync_copy(epk.at[wid], idx_v)

    for p in range(HEADS // 2):
        def dump(p=p):
            pltpu.sync_copy(out_sh.at[stripe], out.at[p, cid, stripe])
            pltpu.sync_copy(d0_sh.at[stripe], den.at[p, cid, 0, stripe])
            pltpu.sync_copy(d1_sh.at[stripe], den.at[p, cid, 1, stripe])
        _sc_pass(hs[p], ts[2 * p], ts[HEADS + 2 * p],
                 ts[2 * p + 1], ts[HEADS + 2 * p + 1], dump,
                 idx_v, eb_v, sc_v, al_v, rows_v,
                 out_sh, d0_sh, d1_sh, sems, zb, zs, stripe, nh)


def _sc_body2(hh, as0, ad0, as1, ad1, epk, zb, zs, out, den,
              idx_v, eb_v, sc_v, al_v, rows_v,
              out_sh, d0_sh, d1_sh, semA, semB):
    cid = lax.axis_index("c")
    sid = lax.axis_index("s")
    wid = sid * NC + cid
    stripe = pl.ds(sid * STR, STR)

    pltpu.sync_copy(epk.at[wid], idx_v)
    nh = jnp.where(cid == 0, S0 // 2, S1 // 2)

    def dump():
        pltpu.sync_copy(out_sh.at[stripe], out.at[cid, stripe])
        pltpu.sync_copy(d0_sh.at[stripe], den.at[cid, 0, stripe])
        pltpu.sync_copy(d1_sh.at[stripe], den.at[cid, 1, stripe])

    _sc_pass(hh, as0, ad0, as1, ad1, dump,
             idx_v, eb_v, sc_v, al_v, rows_v,
             out_sh, d0_sh, d1_sh, (semA, semB), zb, zs, stripe, nh)


_SC_SCRATCH = [
    pltpu.VMEM((SMX, K), jnp.int32),
    pltpu.VMEM((8, K), jnp.int32),
    pltpu.VMEM((8, K), jnp.float32),
    pltpu.VMEM((4, K), jnp.float32),
    pltpu.VMEM((2, K, W), jnp.float32),
    pltpu.VMEM_SHARED((NP, W), jnp.float32),
    pltpu.VMEM_SHARED((NP,), jnp.float32),
    pltpu.VMEM_SHARED((NP,), jnp.float32),
    pltpu.SemaphoreType.DMA,
    pltpu.SemaphoreType.DMA,
]

_sc_gat1 = functools.partial(
    pl.kernel,
    out_type=(jax.ShapeDtypeStruct((HEADS // 2, NC, NP, W), jnp.float32),
              jax.ShapeDtypeStruct((HEADS // 2, NC, 2, NP), jnp.float32)),
    mesh=plsc.VectorSubcoreMesh(core_axis_name="c", subcore_axis_name="s",
                                num_cores=NC, num_subcores=NS),
    compiler_params=pltpu.CompilerParams(needs_layout_passes=False),
    scratch_types=_SC_SCRATCH,
)(_sc_body)

_sc_gat2 = functools.partial(
    pl.kernel,
    out_type=(jax.ShapeDtypeStruct((NC, NP, W), jnp.float32),
              jax.ShapeDtypeStruct((NC, 2, NP), jnp.float32)),
    mesh=plsc.VectorSubcoreMesh(core_axis_name="c", subcore_axis_name="s",
                                num_cores=NC, num_subcores=NS),
    compiler_params=pltpu.CompilerParams(needs_layout_passes=False),
    scratch_types=_SC_SCRATCH,
)(_sc_body2)


def _k3_body(o_ref, d_ref, b1_ref, w_ref, out_ref):
    i = pl.program_id(0)
    acc = jnp.zeros((256, 128), jnp.float32)
    for h in range(HEADS):
        p, q = h // 2, h % 2
        v = (o_ref[p, 0, :, q * HID:(q + 1) * HID]
             + o_ref[p, 1, :, q * HID:(q + 1) * HID])
        dh = d_ref[p, :, q, pl.ds(i * 256, 256)]
        dd = dh[0] + dh[1] + 1e-16
        v = v / dd[:, None] + b1_ref[h]
        v = jnp.where(v > 0.0, v, jnp.exp(v) - 1.0)
        acc = acc + _dot(v, w_ref[h])
    out_ref[...] = acc


def _combine_l2(out1, den1, b1r, w2cat):
    NPAIR = HEADS // 2
    return pl.pallas_call(
        _k3_body,
        grid=(RB,),
        in_specs=[
            pl.BlockSpec((NPAIR, NC, 256, W), lambda i: (0, 0, i, 0)),
            pl.BlockSpec((NPAIR, NC, 2, NP), lambda i: (0, 0, 0, 0)),
            pl.BlockSpec((HEADS, HID), lambda i: (0, 0)),
            pl.BlockSpec((HEADS, HID, 128), lambda i: (0, 0, 0)),
        ],
        out_specs=pl.BlockSpec((256, 128), lambda i: (i, 0)),
        out_shape=jax.ShapeDtypeStruct((NP, 128), jnp.float32),
    )(out1, den1, b1r, w2cat)


def _k4_body(o_ref, d_ref, b_ref, b2_ref, fcw_ref, fcb_ref, out_ref, sums, counts):
    i = pl.program_id(0)
    p = o_ref[0, :, :HID] + o_ref[1, :, :HID]
    dh = d_ref[:, 0, pl.ds(i * 256, 256)]
    dd = dh[0] + dh[1] + 1e-16
    v = p / dd[:, None] + b2_ref[0]
    v = jnp.where(v > 0.0, v, jnp.exp(v) - 1.0)
    bb = b_ref[0, 0]
    oh = (bb[:, None] == lax.broadcasted_iota(jnp.int32, (256, G), 1)).astype(jnp.float32)
    ps = jax.lax.dot_general(oh, v, (((0,), (0,)), ((), ())),
                             precision=_HI, preferred_element_type=jnp.float32)
    pc = jnp.sum(oh, axis=0)

    @pl.when(i == 0)
    def _():
        sums[...] = jnp.zeros_like(sums)
        counts[...] = jnp.zeros_like(counts)

    sums[...] += ps
    counts[...] += pc[None, :]

    @pl.when(i == RB - 1)
    def _():
        c = jnp.maximum(counts[0, :], 1.0)
        pooled = sums[...] / c[:, None]
        out_ref[...] = _dot(pooled, fcw_ref[...]) + fcb_ref[0]


def _pool_fc(out2, den2, batchr, b2r, fcw, fcb):
    return pl.pallas_call(
        _k4_body,
        grid=(RB,),
        in_specs=[
            pl.BlockSpec((NC, 256, W), lambda i: (0, i, 0)),
            pl.BlockSpec((NC, 2, NP), lambda i: (0, 0, 0)),
            pl.BlockSpec((1, 1, 256), lambda i: (i, 0, 0)),
            pl.BlockSpec((1, HID), lambda i: (0, 0)),
            pl.BlockSpec((HID, 128), lambda i: (0, 0)),
            pl.BlockSpec((1, 128), lambda i: (0, 0)),
        ],
        out_specs=pl.BlockSpec((G, 128), lambda i: (0, 0)),
        out_shape=jax.ShapeDtypeStruct((G, 128), jnp.float32),
        scratch_shapes=[pltpu.VMEM((G, HID), jnp.float32),
                        pltpu.VMEM((1, G), jnp.float32)],
    )(out2, den2, batchr, b2r, fcw, fcb)


def kernel(x, edge_index, batch, W1, att_src1, att_dst1, b1,
           W2, att_src2, att_dst2, b2, fc_w, fc_b):
    f32 = jnp.float32
    w1r = W1.reshape(FIN, HEADS // 2, W).transpose(1, 0, 2)
    w1s = jnp.einsum("fhc,hc->fh", W1.reshape(FIN, HEADS, HID), att_src1)
    w1d = jnp.einsum("fhc,hc->fh", W1.reshape(FIN, HEADS, HID), att_dst1)
    w1sd = jnp.concatenate([w1s, w1d], axis=1)
    w2r = W2.reshape(HEADS, HID, HID)
    w2s = (W2 @ att_src2[0]).reshape(HEADS, HID, 1)
    w2d = (W2 @ att_dst2[0]).reshape(HEADS, HID, 1)
    w2cat = jnp.concatenate(
        [w2r, w2s, w2d, jnp.zeros((HEADS, HID, 128 - HID - 2), f32)], axis=2)
    b1r = b1.reshape(HEADS, HID)
    b2r = b2.reshape(1, HID)
    fcw = jnp.zeros((HID, 128), f32).at[:, :2].set(fc_w)
    fcb = jnp.zeros((1, 128), f32).at[:, :2].set(fc_b)

    xp = jnp.pad(x, ((0, NP - N), (0, 0)))
    loop = jnp.arange(N, dtype=jnp.int32)
    pad = jnp.full((ET_PAD - E_TOT,), DUMMY, jnp.int32)
    srcs = jnp.concatenate([edge_index[0], loop, pad])
    dsts = jnp.concatenate([edge_index[1], loop, pad])
    flat = dsts * PACK + srcs
    dfill = jnp.full((SMX * K,), DUMMY * PACK + DUMMY, jnp.int32)
    rows_list, off = [], 0
    for w in range(NW):
        lw = (S0 if w % NC == 0 else S1) * K
        seg = flat[off:off + lw]
        off += lw
        rows_list.append(jnp.concatenate([seg, dfill[:SMX * K - lw]]))
    epk = jnp.stack(rows_list).reshape(NW, SMX, K)
    batchr = jnp.concatenate(
        [batch, jnp.full((NP - N,), G, jnp.int32)]).reshape(RB, 1, 256)
    zb = jnp.zeros((NP, W), f32)
    zs = jnp.zeros((NP,), f32)

    h4 = _mm_heads(xp, w1r)
    asad = _mm_asad(xp, w1sd)
    asadt = asad.T

    hs = [h4[p] for p in range(HEADS // 2)]
    ts = [asadt[i] for i in range(2 * HEADS)]
    out1, den1 = _sc_gat1(*hs, *ts, epk, zb, zs)

    o2pre = _combine_l2(out1, den1, b1r, w2cat)
    as2 = o2pre[:, HID]
    ad2 = o2pre[:, HID + 1]

    out2, den2 = _sc_gat2(o2pre, as2, ad2, zs, zs, epk, zb, zs)

    logits = _pool_fc(out2, den2, batchr, b2r, fcw, fcb)
    return logits[:, :2]


# speedup vs baseline: 1.6891x; 1.5089x over previous
import functools
import jax
import jax.numpy as jnp
from jax import lax
from jax.experimental import pallas as pl
from jax.experimental.pallas import tpu as pltpu
from jax.experimental.pallas import tpu_sc as plsc

N = 10000
FIN = 128
HID = 64
HEADS = 8
G = 64

NP = 10240
DUMMY = N

NC, NS, L = 2, 16, 16
NW = NC * NS
K = 128
E_TOT = 320000 + N
S0, S1 = 96, 66
SMX = max(S0, S1)
ET_PAD = NS * K * (S0 + S1)
PACK = 16384
RB = NP // 256
STR = NP // NS
W = 2 * HID

_HI = jax.lax.Precision.HIGHEST


def _dot(a, b):
    return jax.lax.dot_general(a, b, (((1,), (0,)), ((), ())),
                               precision=_HI, preferred_element_type=jnp.float32)


def _k1a_body(x_ref, w_ref, h_ref):
    h_ref[0] = _dot(x_ref[...], w_ref[0])


def _mm_heads(xp, w1r):
    return pl.pallas_call(
        _k1a_body,
        grid=(HEADS // 2, RB),
        in_specs=[
            pl.BlockSpec((256, FIN), lambda p, i: (i, 0)),
            pl.BlockSpec((1, FIN, W), lambda p, i: (p, 0, 0)),
        ],
        out_specs=pl.BlockSpec((1, 256, W), lambda p, i: (p, i, 0)),
        out_shape=jax.ShapeDtypeStruct((HEADS // 2, NP, W), jnp.float32),
    )(xp, w1r)


def _k1b_body(x_ref, w_ref, o_ref):
    o_ref[...] = _dot(x_ref[...], w_ref[...])


def _mm_asad(xp, w1sd):
    return pl.pallas_call(
        _k1b_body,
        grid=(RB,),
        in_specs=[
            pl.BlockSpec((256, FIN), lambda i: (i, 0)),
            pl.BlockSpec((FIN, 2 * HEADS), lambda i: (0, 0)),
        ],
        out_specs=pl.BlockSpec((256, 2 * HEADS), lambda i: (i, 0)),
        out_shape=jax.ShapeDtypeStruct((NP, 2 * HEADS), jnp.float32),
    )(xp, w1sd)


def _sc_pass(hh, as0, ad0, as1, ad1, dump, idx_v, eb_v, sc_v, al_v, rows_v,
             out_sh, d0_sh, d1_sh, sems, zb, zs, stripe, nh):
    pltpu.sync_copy(zb.at[stripe], out_sh.at[stripe])
    pltpu.sync_copy(zs.at[stripe], d0_sh.at[stripe])
    pltpu.sync_copy(zs.at[stripe], d1_sh.at[stripe])

    plsc.subcore_barrier()

    def issue(t, b):
        for g in range(K // L):
            gs = pl.ds(g * L, L)
            pk = idx_v[t, gs]
            eb_v[2 * b, gs] = pk & (PACK - 1)
            eb_v[2 * b + 1, gs] = lax.shift_right_logical(pk, 14)
        src_r = eb_v.at[2 * b]
        dst_r = eb_v.at[2 * b + 1]
        pltpu.async_copy(as0.at[src_r], sc_v.at[4 * b + 0], sems[b])
        pltpu.async_copy(ad0.at[dst_r], sc_v.at[4 * b + 1], sems[b])
        pltpu.async_copy(as1.at[src_r], sc_v.at[4 * b + 2], sems[b])
        pltpu.async_copy(ad1.at[dst_r], sc_v.at[4 * b + 3], sems[b])
        pltpu.async_copy(hh.at[src_r], rows_v.at[b], sems[b])

    def wait(b):
        pltpu.make_async_copy(as0.at[pl.ds(0, K)], sc_v.at[4 * b + 0], sems[b]).wait()
        pltpu.make_async_copy(ad0.at[pl.ds(0, K)], sc_v.at[4 * b + 1], sems[b]).wait()
        pltpu.make_async_copy(as1.at[pl.ds(0, K)], sc_v.at[4 * b + 2], sems[b]).wait()
        pltpu.make_async_copy(ad1.at[pl.ds(0, K)], sc_v.at[4 * b + 3], sems[b]).wait()
        pltpu.make_async_copy(hh.at[pl.ds(0, K)], rows_v.at[b], sems[b]).wait()

    def process(t, b):
        for g in range(K // L):
            gs = pl.ds(g * L, L)
            z0 = sc_v[4 * b + 0, gs] + sc_v[4 * b + 1, gs]
            z0 = jnp.where(z0 >= 0.0, z0, 0.2 * z0)
            al_v[2 * b, gs] = jnp.exp(z0)
            z1 = sc_v[4 * b + 2, gs] + sc_v[4 * b + 3, gs]
            z1 = jnp.where(z1 >= 0.0, z1, 0.2 * z1)
            al_v[2 * b + 1, gs] = jnp.exp(z1)
        def scale(e, c2):
            av0 = plsc.load_gather(al_v.at[2 * b], [jnp.full((L,), e, jnp.int32)])
            av1 = plsc.load_gather(al_v.at[2 * b + 1], [jnp.full((L,), e, jnp.int32)])
            for c in range(HID // L):
                rows_v[b, e, pl.ds(c * L, L)] = \
                    rows_v[b, e, pl.ds(c * L, L)] * av0
                rows_v[b, e, pl.ds(HID + c * L, L)] = \
                    rows_v[b, e, pl.ds(HID + c * L, L)] * av1
            return c2
        lax.fori_loop(0, K, scale, 0, unroll=4)
        for g in range(K // L):
            gs = pl.ds(g * L, L)
            eb_v[4, gs] = lax.shift_right_logical(idx_v[t, gs], 14)
        dsc = eb_v.at[4]
        pltpu.sync_copy(rows_v.at[b], out_sh.at[dsc], add=True)
        pltpu.sync_copy(al_v.at[2 * b], d0_sh.at[dsc], add=True)
        pltpu.sync_copy(al_v.at[2 * b + 1], d1_sh.at[dsc], add=True)

    issue(0, 0)
    issue(1, 1)

    def pair(tt, carry):
        t0 = 2 * tt
        wait(0)
        process(t0, 0)

        @pl.when(tt < nh - 1)
        def _():
            issue(t0 + 2, 0)

        wait(1)
        process(t0 + 1, 1)

        @pl.when(tt < nh - 1)
        def _():
            issue(t0 + 3, 1)

        return carry

    lax.fori_loop(0, nh, pair, 0)

    plsc.subcore_barrier()
    dump()
    plsc.subcore_barrier()


def _sc_body(h0, h1, h2, h3,
             t0, t1, t2, t3, t4, t5, t6, t7,
             t8, t9, t10, t11, t12, t13, t14, t15,
             epk, zb, zs, out, den,
             idx_v, eb_v, sc_v, al_v, rows_v,
             out_sh, d0_sh, d1_sh, semA, semB):
    cid = lax.axis_index("c")
    sid = lax.axis_index("s")
    wid = sid * NC + cid
    stripe = pl.ds(sid * STR, STR)
    sems = (semA, semB)
    hs = (h0, h1, h2, h3)
    ts = (t0, t1, t2, t3, t4, t5, t6, t7,
          t8, t9, t10, t11, t12, t13, t14, t15)
    nh = jnp.where(cid == 0, S0 // 2, S1 // 2)

    pltpu.sync_copy(epk.at[wid], idx_v)

    for p in range(HEADS // 2):
        def dump(p=p):
            pltpu.sync_copy(out_sh.at[stripe], out.at[p, cid, stripe])
            pltpu.sync_copy(d0_sh.at[stripe], den.at[p, cid, 0, stripe])
            pltpu.sync_copy(d1_sh.at[stripe], den.at[p, cid, 1, stripe])
        _sc_pass(hs[p], ts[2 * p], ts[HEADS + 2 * p],
                 ts[2 * p + 1], ts[HEADS + 2 * p + 1], dump,
                 idx_v, eb_v, sc_v, al_v, rows_v,
                 out_sh, d0_sh, d1_sh, sems, zb, zs, stripe, nh)


def _sc_body2(hh, as0, ad0, as1, ad1, epk, zb, zs, out, den,
              idx_v, eb_v, sc_v, al_v, rows_v,
              out_sh, d0_sh, d1_sh, semA, semB):
    cid = lax.axis_index("c")
    sid = lax.axis_index("s")
    wid = sid * NC + cid
    stripe = pl.ds(sid * STR, STR)

    pltpu.sync_copy(epk.at[wid], idx_v)
    nh = jnp.where(cid == 0, S0 // 2, S1 // 2)

    def dump():
        pltpu.sync_copy(out_sh.at[stripe], out.at[cid, stripe])
        pltpu.sync_copy(d0_sh.at[stripe], den.at[cid, 0, stripe])
        pltpu.sync_copy(d1_sh.at[stripe], den.at[cid, 1, stripe])

    _sc_pass(hh, as0, ad0, as1, ad1, dump,
             idx_v, eb_v, sc_v, al_v, rows_v,
             out_sh, d0_sh, d1_sh, (semA, semB), zb, zs, stripe, nh)


_SC_SCRATCH = [
    pltpu.VMEM((SMX, K), jnp.int32),
    pltpu.VMEM((8, K), jnp.int32),
    pltpu.VMEM((8, K), jnp.float32),
    pltpu.VMEM((4, K), jnp.float32),
    pltpu.VMEM((2, K, W), jnp.float32),
    pltpu.VMEM_SHARED((NP, W), jnp.float32),
    pltpu.VMEM_SHARED((NP,), jnp.float32),
    pltpu.VMEM_SHARED((NP,), jnp.float32),
    pltpu.SemaphoreType.DMA,
    pltpu.SemaphoreType.DMA,
]

_sc_gat1 = functools.partial(
    pl.kernel,
    out_type=(jax.ShapeDtypeStruct((HEADS // 2, NC, NP, W), jnp.float32),
              jax.ShapeDtypeStruct((HEADS // 2, NC, 2, NP), jnp.float32)),
    mesh=plsc.VectorSubcoreMesh(core_axis_name="c", subcore_axis_name="s",
                                num_cores=NC, num_subcores=NS),
    compiler_params=pltpu.CompilerParams(needs_layout_passes=False),
    scratch_types=_SC_SCRATCH,
)(_sc_body)

_sc_gat2 = functools.partial(
    pl.kernel,
    out_type=(jax.ShapeDtypeStruct((NC, NP, W), jnp.float32),
              jax.ShapeDtypeStruct((NC, 2, NP), jnp.float32)),
    mesh=plsc.VectorSubcoreMesh(core_axis_name="c", subcore_axis_name="s",
                                num_cores=NC, num_subcores=NS),
    compiler_params=pltpu.CompilerParams(needs_layout_passes=False),
    scratch_types=_SC_SCRATCH,
)(_sc_body2)


def _k3_body(o_ref, d_ref, b1_ref, w_ref, out_ref):
    i = pl.program_id(0)
    acc = jnp.zeros((256, 128), jnp.float32)
    for h in range(HEADS):
        p, q = h // 2, h % 2
        v = (o_ref[p, 0, :, q * HID:(q + 1) * HID]
             + o_ref[p, 1, :, q * HID:(q + 1) * HID])
        dh = d_ref[p, :, q, pl.ds(i * 256, 256)]
        dd = dh[0] + dh[1] + 1e-16
        v = v / dd[:, None] + b1_ref[h]
        v = jnp.where(v > 0.0, v, jnp.exp(v) - 1.0)
        acc = acc + _dot(v, w_ref[h])
    out_ref[...] = acc


def _combine_l2(out1, den1, b1r, w2cat):
    NPAIR = HEADS // 2
    return pl.pallas_call(
        _k3_body,
        grid=(RB,),
        in_specs=[
            pl.BlockSpec((NPAIR, NC, 256, W), lambda i: (0, 0, i, 0)),
            pl.BlockSpec((NPAIR, NC, 2, NP), lambda i: (0, 0, 0, 0)),
            pl.BlockSpec((HEADS, HID), lambda i: (0, 0)),
            pl.BlockSpec((HEADS, HID, 128), lambda i: (0, 0, 0)),
        ],
        out_specs=pl.BlockSpec((256, 128), lambda i: (i, 0)),
        out_shape=jax.ShapeDtypeStruct((NP, 128), jnp.float32),
    )(out1, den1, b1r, w2cat)


def _k4_body(o_ref, d_ref, b_ref, b2_ref, fcw_ref, fcb_ref, out_ref, sums, counts):
    i = pl.program_id(0)
    p = o_ref[0, :, :HID] + o_ref[1, :, :HID]
    dh = d_ref[:, 0, pl.ds(i * 256, 256)]
    dd = dh[0] + dh[1] + 1e-16
    v = p / dd[:, None] + b2_ref[0]
    v = jnp.where(v > 0.0, v, jnp.exp(v) - 1.0)
    bb = b_ref[0, 0]
    oh = (bb[:, None] == lax.broadcasted_iota(jnp.int32, (256, G), 1)).astype(jnp.float32)
    ps = jax.lax.dot_general(oh, v, (((0,), (0,)), ((), ())),
                             precision=_HI, preferred_element_type=jnp.float32)
    pc = jnp.sum(oh, axis=0)

    @pl.when(i == 0)
    def _():
        sums[...] = jnp.zeros_like(sums)
        counts[...] = jnp.zeros_like(counts)

    sums[...] += ps
    counts[...] += pc[None, :]

    @pl.when(i == RB - 1)
    def _():
        c = jnp.maximum(counts[0, :], 1.0)
        pooled = sums[...] / c[:, None]
        out_ref[...] = _dot(pooled, fcw_ref[...]) + fcb_ref[0]


def _pool_fc(out2, den2, batchr, b2r, fcw, fcb):
    return pl.pallas_call(
        _k4_body,
        grid=(RB,),
        in_specs=[
            pl.BlockSpec((NC, 256, W), lambda i: (0, i, 0)),
            pl.BlockSpec((NC, 2, NP), lambda i: (0, 0, 0)),
            pl.BlockSpec((1, 1, 256), lambda i: (i, 0, 0)),
            pl.BlockSpec((1, HID), lambda i: (0, 0)),
            pl.BlockSpec((HID, 128), lambda i: (0, 0)),
            pl.BlockSpec((1, 128), lambda i: (0, 0)),
        ],
        out_specs=pl.BlockSpec((G, 128), lambda i: (0, 0)),
        out_shape=jax.ShapeDtypeStruct((G, 128), jnp.float32),
        scratch_shapes=[pltpu.VMEM((G, HID), jnp.float32),
                        pltpu.VMEM((1, G), jnp.float32)],
    )(out2, den2, batchr, b2r, fcw, fcb)


def kernel(x, edge_index, batch, W1, att_src1, att_dst1, b1,
           W2, att_src2, att_dst2, b2, fc_w, fc_b):
    f32 = jnp.float32
    w1r = W1.reshape(FIN, HEADS // 2, W).transpose(1, 0, 2)
    w1s = jnp.einsum("fhc,hc->fh", W1.reshape(FIN, HEADS, HID), att_src1)
    w1d = jnp.einsum("fhc,hc->fh", W1.reshape(FIN, HEADS, HID), att_dst1)
    w1sd = jnp.concatenate([w1s, w1d], axis=1)
    w2r = W2.reshape(HEADS, HID, HID)
    w2s = (W2 @ att_src2[0]).reshape(HEADS, HID, 1)
    w2d = (W2 @ att_dst2[0]).reshape(HEADS, HID, 1)
    w2cat = jnp.concatenate(
        [w2r, w2s, w2d, jnp.zeros((HEADS, HID, 128 - HID - 2), f32)], axis=2)
    b1r = b1.reshape(HEADS, HID)
    b2r = b2.reshape(1, HID)
    fcw = jnp.zeros((HID, 128), f32).at[:, :2].set(fc_w)
    fcb = jnp.zeros((1, 128), f32).at[:, :2].set(fc_b)

    xp = jnp.pad(x, ((0, NP - N), (0, 0)))
    loop = jnp.arange(N, dtype=jnp.int32)
    pad = jnp.full((ET_PAD - E_TOT,), DUMMY, jnp.int32)
    srcs = jnp.concatenate([edge_index[0], loop, pad])
    dsts = jnp.concatenate([edge_index[1], loop, pad])
    flat = dsts * PACK + srcs
    dfill = jnp.full((SMX * K,), DUMMY * PACK + DUMMY, jnp.int32)
    rows_list, off = [], 0
    for w in range(NW):
        lw = (S0 if w % NC == 0 else S1) * K
        seg = flat[off:off + lw]
        off += lw
        rows_list.append(jnp.concatenate([seg, dfill[:SMX * K - lw]]))
    epk = jnp.stack(rows_list).reshape(NW, SMX, K)
    batchr = jnp.concatenate(
        [batch, jnp.full((NP - N,), G, jnp.int32)]).reshape(RB, 1, 256)
    zb = jnp.zeros((NP, W), f32)
    zs = jnp.zeros((NP,), f32)

    h4 = _mm_heads(xp, w1r)
    asad = _mm_asad(xp, w1sd)
    asadt = asad.T

    hs = [h4[p] for p in range(HEADS // 2)]
    ts = [asadt[i] for i in range(2 * HEADS)]
    out1, den1 = _sc_gat1(*hs, *ts, epk, zb, zs)

    o2pre = _combine_l2(out1, den1, b1r, w2cat)
    as2 = o2pre[:, HID]
    ad2 = o2pre[:, HID + 1]

    out2, den2 = _sc_gat2(o2pre, as2, ad2, zs, zs, epk, zb, zs)

    logits = _pool_fc(out2, den2, batchr, b2r, fcw, fcb)
    return logits[:, :2]


# speedup vs baseline: 1.7739x; 1.0502x over previous
import functools
import jax
import jax.numpy as jnp
from jax import lax
from jax.experimental import pallas as pl
from jax.experimental.pallas import tpu as pltpu
from jax.experimental.pallas import tpu_sc as plsc

N = 10000
FIN = 128
HID = 64
HEADS = 8
G = 64

NP = 10240
DUMMY = N

NC, NS, L = 2, 16, 16
NW = NC * NS
K = 128
E_TOT = 320000 + N
S0, S1 = 96, 66
SMX = max(S0, S1)
ET_PAD = NS * K * (S0 + S1)
PACK = 16384
RB = NP // 256
STR = NP // NS
W = 2 * HID

_HI = jax.lax.Precision.HIGHEST


def _dot(a, b):
    return jax.lax.dot_general(a, b, (((1,), (0,)), ((), ())),
                               precision=_HI, preferred_element_type=jnp.float32)


def _k1a_body(x_ref, w_ref, h_ref):
    h_ref[0] = _dot(x_ref[...], w_ref[0])


def _mm_heads(xp, w1r):
    return pl.pallas_call(
        _k1a_body,
        grid=(HEADS // 2, RB),
        in_specs=[
            pl.BlockSpec((256, FIN), lambda p, i: (i, 0)),
            pl.BlockSpec((1, FIN, W), lambda p, i: (p, 0, 0)),
        ],
        out_specs=pl.BlockSpec((1, 256, W), lambda p, i: (p, i, 0)),
        out_shape=jax.ShapeDtypeStruct((HEADS // 2, NP, W), jnp.float32),
    )(xp, w1r)


def _k1b_body(x_ref, w_ref, o_ref):
    o_ref[...] = _dot(x_ref[...], w_ref[...])


def _mm_asad(xp, w1sd):
    return pl.pallas_call(
        _k1b_body,
        grid=(RB,),
        in_specs=[
            pl.BlockSpec((256, FIN), lambda i: (i, 0)),
            pl.BlockSpec((FIN, 2 * HEADS), lambda i: (0, 0)),
        ],
        out_specs=pl.BlockSpec((256, 2 * HEADS), lambda i: (i, 0)),
        out_shape=jax.ShapeDtypeStruct((NP, 2 * HEADS), jnp.float32),
    )(xp, w1sd)


def _sc_pass(hh, as0, ad0, as1, ad1, dump, idx_v, eb_v, sc_v, al_v, rows_v,
             out_sh, d0_sh, d1_sh, sems, zb, zs, stripe, nh):
    pltpu.sync_copy(zb.at[stripe], out_sh.at[stripe])
    pltpu.sync_copy(zs.at[stripe], d0_sh.at[stripe])
    pltpu.sync_copy(zs.at[stripe], d1_sh.at[stripe])

    plsc.subcore_barrier()

    def issue(t, b):
        for g in range(K // L):
            gs = pl.ds(g * L, L)
            pk = idx_v[t, gs]
            eb_v[2 * b, gs] = pk & (PACK - 1)
            eb_v[2 * b + 1, gs] = lax.shift_right_logical(pk, 14)
        src_r = eb_v.at[2 * b]
        dst_r = eb_v.at[2 * b + 1]
        pltpu.async_copy(as0.at[src_r], sc_v.at[4 * b + 0], sems[b])
        pltpu.async_copy(ad0.at[dst_r], sc_v.at[4 * b + 1], sems[b])
        pltpu.async_copy(as1.at[src_r], sc_v.at[4 * b + 2], sems[b])
        pltpu.async_copy(ad1.at[dst_r], sc_v.at[4 * b + 3], sems[b])
        pltpu.async_copy(hh.at[src_r], rows_v.at[b], sems[b])

    def wait(b):
        pltpu.make_async_copy(as0.at[pl.ds(0, K)], sc_v.at[4 * b + 0], sems[b]).wait()
        pltpu.make_async_copy(ad0.at[pl.ds(0, K)], sc_v.at[4 * b + 1], sems[b]).wait()
        pltpu.make_async_copy(as1.at[pl.ds(0, K)], sc_v.at[4 * b + 2], sems[b]).wait()
        pltpu.make_async_copy(ad1.at[pl.ds(0, K)], sc_v.at[4 * b + 3], sems[b]).wait()
        pltpu.make_async_copy(hh.at[pl.ds(0, K)], rows_v.at[b], sems[b]).wait()

    def process(t, b):
        for g in range(K // L):
            gs = pl.ds(g * L, L)
            z0 = sc_v[4 * b + 0, gs] + sc_v[4 * b + 1, gs]
            z0 = jnp.where(z0 >= 0.0, z0, 0.2 * z0)
            al_v[2 * b, gs] = jnp.exp(z0)
            z1 = sc_v[4 * b + 2, gs] + sc_v[4 * b + 3, gs]
            z1 = jnp.where(z1 >= 0.0, z1, 0.2 * z1)
            al_v[2 * b + 1, gs] = jnp.exp(z1)
        def scale(e, c2):
            av0 = plsc.load_gather(al_v.at[2 * b], [jnp.full((L,), e, jnp.int32)])
            av1 = plsc.load_gather(al_v.at[2 * b + 1], [jnp.full((L,), e, jnp.int32)])
            for c in range(HID // L):
                rows_v[b, e, pl.ds(c * L, L)] = \
                    rows_v[b, e, pl.ds(c * L, L)] * av0
                rows_v[b, e, pl.ds(HID + c * L, L)] = \
                    rows_v[b, e, pl.ds(HID + c * L, L)] * av1
            return c2
        lax.fori_loop(0, K, scale, 0, unroll=4)
        for g in range(K // L):
            gs = pl.ds(g * L, L)
            eb_v[4, gs] = lax.shift_right_logical(idx_v[t, gs], 14)
        dsc = eb_v.at[4]
        pltpu.sync_copy(rows_v.at[b], out_sh.at[dsc], add=True)
        pltpu.sync_copy(al_v.at[2 * b], d0_sh.at[dsc], add=True)
        pltpu.sync_copy(al_v.at[2 * b + 1], d1_sh.at[dsc], add=True)

    issue(0, 0)
    issue(1, 1)

    def pair(tt, carry):
        t0 = 2 * tt
        wait(0)
        process(t0, 0)

        @pl.when(tt < nh - 1)
        def _():
            issue(t0 + 2, 0)

        wait(1)
        process(t0 + 1, 1)

        @pl.when(tt < nh - 1)
        def _():
            issue(t0 + 3, 1)

        return carry

    lax.fori_loop(0, nh, pair, 0)

    plsc.subcore_barrier()
    dump()
    plsc.subcore_barrier()


def _sc_body(h0, h1, h2, h3,
             t0, t1, t2, t3, t4, t5, t6, t7,
             t8, t9, t10, t11, t12, t13, t14, t15,
             epk, zb, zs, out, den,
             idx_v, eb_v, sc_v, al_v, rows_v,
             out_sh, d0_sh, d1_sh, semA, semB):
    cid = lax.axis_index("c")
    sid = lax.axis_index("s")
    wid = sid * NC + cid
    stripe = pl.ds(sid * STR, STR)
    sems = (semA, semB)
    hs = (h0, h1, h2, h3)
    ts = (t0, t1, t2, t3, t4, t5, t6, t7,
          t8, t9, t10, t11, t12, t13, t14, t15)
    nh = jnp.where(cid == 0, S0 // 2, S1 // 2)

    pltpu.sync_copy(epk.at[wid], idx_v)

    for p in range(HEADS // 2):
        def dump(p=p):
            pltpu.sync_copy(out_sh.at[stripe], out.at[p, cid, stripe])
            pltpu.sync_copy(d0_sh.at[stripe], den.at[p, cid, 0, stripe])
            pltpu.sync_copy(d1_sh.at[stripe], den.at[p, cid, 1, stripe])
        _sc_pass(hs[p], ts[2 * p], ts[HEADS + 2 * p],
                 ts[2 * p + 1], ts[HEADS + 2 * p + 1], dump,
                 idx_v, eb_v, sc_v, al_v, rows_v,
                 out_sh, d0_sh, d1_sh, sems, zb, zs, stripe, nh)


def _sc_body2(hh, as0, ad0, as1, ad1, epk, zb, zs, out, den,
              idx_v, eb_v, sc_v, al_v, rows_v,
              out_sh, d0_sh, d1_sh, semA, semB):
    cid = lax.axis_index("c")
    sid = lax.axis_index("s")
    wid = sid * NC + cid
    stripe = pl.ds(sid * STR, STR)

    pltpu.sync_copy(epk.at[wid], idx_v)
    nh = jnp.where(cid == 0, S0 // 2, S1 // 2)

    def dump():
        pltpu.sync_copy(out_sh.at[stripe], out.at[cid, stripe])
        pltpu.sync_copy(d0_sh.at[stripe], den.at[cid, 0, stripe])
        pltpu.sync_copy(d1_sh.at[stripe], den.at[cid, 1, stripe])

    _sc_pass(hh, as0, ad0, as1, ad1, dump,
             idx_v, eb_v, sc_v, al_v, rows_v,
             out_sh, d0_sh, d1_sh, (semA, semB), zb, zs, stripe, nh)


_SC_SCRATCH = [
    pltpu.VMEM((SMX, K), jnp.int32),
    pltpu.VMEM((8, K), jnp.int32),
    pltpu.VMEM((8, K), jnp.float32),
    pltpu.VMEM((4, K), jnp.float32),
    pltpu.VMEM((2, K, W), jnp.float32),
    pltpu.VMEM_SHARED((NP, W), jnp.float32),
    pltpu.VMEM_SHARED((NP,), jnp.float32),
    pltpu.VMEM_SHARED((NP,), jnp.float32),
    pltpu.SemaphoreType.DMA,
    pltpu.SemaphoreType.DMA,
]

_sc_gat1 = functools.partial(
    pl.kernel,
    out_type=(jax.ShapeDtypeStruct((HEADS // 2, NC, NP, W), jnp.float32),
              jax.ShapeDtypeStruct((HEADS // 2, NC, 2, NP), jnp.float32)),
    mesh=plsc.VectorSubcoreMesh(core_axis_name="c", subcore_axis_name="s",
                                num_cores=NC, num_subcores=NS),
    compiler_params=pltpu.CompilerParams(needs_layout_passes=False),
    scratch_types=_SC_SCRATCH,
)(_sc_body)

_sc_gat2 = functools.partial(
    pl.kernel,
    out_type=(jax.ShapeDtypeStruct((NC, NP, W), jnp.float32),
              jax.ShapeDtypeStruct((NC, 2, NP), jnp.float32)),
    mesh=plsc.VectorSubcoreMesh(core_axis_name="c", subcore_axis_name="s",
                                num_cores=NC, num_subcores=NS),
    compiler_params=pltpu.CompilerParams(needs_layout_passes=False),
    scratch_types=_SC_SCRATCH,
)(_sc_body2)


def _k3_body(o_ref, d_ref, b1_ref, w_ref, out_ref):
    i = pl.program_id(0)
    acc = jnp.zeros((256, 128), jnp.float32)
    for h in range(HEADS):
        p, q = h // 2, h % 2
        v = (o_ref[p, 0, :, q * HID:(q + 1) * HID]
             + o_ref[p, 1, :, q * HID:(q + 1) * HID])
        dh = d_ref[p, :, q, pl.ds(i * 256, 256)]
        dd = dh[0] + dh[1] + 1e-16
        v = v / dd[:, None] + b1_ref[h]
        v = jnp.where(v > 0.0, v, jnp.exp(v) - 1.0)
        acc = acc + _dot(v, w_ref[h])
    out_ref[...] = acc


def _combine_l2(out1, den1, b1r, w2cat):
    NPAIR = HEADS // 2
    return pl.pallas_call(
        _k3_body,
        grid=(RB,),
        in_specs=[
            pl.BlockSpec((NPAIR, NC, 256, W), lambda i: (0, 0, i, 0)),
            pl.BlockSpec((NPAIR, NC, 2, NP), lambda i: (0, 0, 0, 0)),
            pl.BlockSpec((HEADS, HID), lambda i: (0, 0)),
            pl.BlockSpec((HEADS, HID, 128), lambda i: (0, 0, 0)),
        ],
        out_specs=pl.BlockSpec((256, 128), lambda i: (i, 0)),
        out_shape=jax.ShapeDtypeStruct((NP, 128), jnp.float32),
    )(out1, den1, b1r, w2cat)


def _k4_body(o_ref, d_ref, b_ref, b2_ref, fcw_ref, fcb_ref, out_ref, sums, counts):
    i = pl.program_id(0)
    p = o_ref[0, :, :HID] + o_ref[1, :, :HID]
    dh = d_ref[:, 0, pl.ds(i * 256, 256)]
    dd = dh[0] + dh[1] + 1e-16
    v = p / dd[:, None] + b2_ref[0]
    v = jnp.where(v > 0.0, v, jnp.exp(v) - 1.0)
    bb = b_ref[0, 0]
    oh = (bb[:, None] == lax.broadcasted_iota(jnp.int32, (256, G), 1)).astype(jnp.float32)
    ps = jax.lax.dot_general(oh, v, (((0,), (0,)), ((), ())),
                             precision=_HI, preferred_element_type=jnp.float32)
    pc = jnp.sum(oh, axis=0)

    @pl.when(i == 0)
    def _():
        sums[...] = jnp.zeros_like(sums)
        counts[...] = jnp.zeros_like(counts)

    sums[...] += ps
    counts[...] += pc[None, :]

    @pl.when(i == RB - 1)
    def _():
        c = jnp.maximum(counts[0, :], 1.0)
        pooled = sums[...] / c[:, None]
        out_ref[...] = _dot(pooled, fcw_ref[...]) + fcb_ref[0]


def _pool_fc(out2, den2, batchr, b2r, fcw, fcb):
    return pl.pallas_call(
        _k4_body,
        grid=(RB,),
        in_specs=[
            pl.BlockSpec((NC, 256, W), lambda i: (0, i, 0)),
            pl.BlockSpec((NC, 2, NP), lambda i: (0, 0, 0)),
            pl.BlockSpec((1, 1, 256), lambda i: (i, 0, 0)),
            pl.BlockSpec((1, HID), lambda i: (0, 0)),
            pl.BlockSpec((HID, 128), lambda i: (0, 0)),
            pl.BlockSpec((1, 128), lambda i: (0, 0)),
        ],
        out_specs=pl.BlockSpec((G, 128), lambda i: (0, 0)),
        out_shape=jax.ShapeDtypeStruct((G, 128), jnp.float32),
        scratch_shapes=[pltpu.VMEM((G, HID), jnp.float32),
                        pltpu.VMEM((1, G), jnp.float32)],
    )(out2, den2, batchr, b2r, fcw, fcb)


def kernel(x, edge_index, batch, W1, att_src1, att_dst1, b1,
           W2, att_src2, att_dst2, b2, fc_w, fc_b):
    f32 = jnp.float32
    w1r = W1.reshape(FIN, HEADS // 2, W).transpose(1, 0, 2)
    w1s = jnp.einsum("fhc,hc->fh", W1.reshape(FIN, HEADS, HID), att_src1)
    w1d = jnp.einsum("fhc,hc->fh", W1.reshape(FIN, HEADS, HID), att_dst1)
    w1sd = jnp.concatenate([w1s, w1d], axis=1)
    w2r = W2.reshape(HEADS, HID, HID)
    w2s = (W2 @ att_src2[0]).reshape(HEADS, HID, 1)
    w2d = (W2 @ att_dst2[0]).reshape(HEADS, HID, 1)
    w2cat = jnp.concatenate(
        [w2r, w2s, w2d, jnp.zeros((HEADS, HID, 128 - HID - 2), f32)], axis=2)
    b1r = b1.reshape(HEADS, HID)
    b2r = b2.reshape(1, HID)
    fcw = jnp.zeros((HID, 128), f32).at[:, :2].set(fc_w)
    fcb = jnp.zeros((1, 128), f32).at[:, :2].set(fc_b)

    xp = jnp.pad(x, ((0, NP - N), (0, 0)))
    loop = jnp.arange(N, dtype=jnp.int32)
    pad = DUMMY + jnp.arange(ET_PAD - E_TOT, dtype=jnp.int32) % (NP - N)
    srcs = jnp.concatenate([edge_index[0], loop, pad])
    dsts = jnp.concatenate([edge_index[1], loop, pad])
    flat = dsts * PACK + srcs
    dfill = (DUMMY + jnp.arange(SMX * K, dtype=jnp.int32) % (NP - N)) * PACK \
        + DUMMY
    rows_list, off = [], 0
    for w in range(NW):
        lw = (S0 if w % NC == 0 else S1) * K
        seg = flat[off:off + lw]
        off += lw
        rows_list.append(jnp.concatenate([seg, dfill[:SMX * K - lw]]))
    epk = jnp.stack(rows_list).reshape(NW, SMX, K)
    batchr = jnp.concatenate(
        [batch, jnp.full((NP - N,), G, jnp.int32)]).reshape(RB, 1, 256)
    zb = jnp.zeros((NP, W), f32)
    zs = jnp.zeros((NP,), f32)

    h4 = _mm_heads(xp, w1r)
    asad = _mm_asad(xp, w1sd)
    asadt = asad.T

    hs = [h4[p] for p in range(HEADS // 2)]
    ts = [asadt[i] for i in range(2 * HEADS)]
    out1, den1 = _sc_gat1(*hs, *ts, epk, zb, zs)

    o2pre = _combine_l2(out1, den1, b1r, w2cat)
    as2 = o2pre[:, HID]
    ad2 = o2pre[:, HID + 1]

    out2, den2 = _sc_gat2(o2pre, as2, ad2, zs, zs, epk, zb, zs)

    logits = _pool_fc(out2, den2, batchr, b2r, fcw, fcb)
    return logits[:, :2]


# speedup vs baseline: 1.9255x; 1.0855x over previous
import functools
import jax
import jax.numpy as jnp
from jax import lax
from jax.experimental import pallas as pl
from jax.experimental.pallas import tpu as pltpu
from jax.experimental.pallas import tpu_sc as plsc

N = 10000
FIN = 128
HID = 64
HEADS = 8
G = 64

NP = 10240
DUMMY = N

NC, NS, L = 2, 16, 16
NW = NC * NS
K = 128
E_TOT = 320000 + N
S0, S1 = 82, 82
SMX = max(S0, S1)
ET_PAD = NS * K * (S0 + S1)
PACK = 16384
RB = NP // 256
STR = NP // NS
W = 2 * HID

_HI = jax.lax.Precision.HIGHEST


def _dot(a, b):
    return jax.lax.dot_general(a, b, (((1,), (0,)), ((), ())),
                               precision=_HI, preferred_element_type=jnp.float32)


def _k1a_body(x_ref, w_ref, h_ref):
    h_ref[0] = _dot(x_ref[...], w_ref[0])


def _mm_heads(xp, w1r):
    return pl.pallas_call(
        _k1a_body,
        grid=(HEADS // 2, RB),
        in_specs=[
            pl.BlockSpec((256, FIN), lambda p, i: (i, 0)),
            pl.BlockSpec((1, FIN, W), lambda p, i: (p, 0, 0)),
        ],
        out_specs=pl.BlockSpec((1, 256, W), lambda p, i: (p, i, 0)),
        out_shape=jax.ShapeDtypeStruct((HEADS // 2, NP, W), jnp.float32),
    )(xp, w1r)


def _k1b_body(x_ref, w_ref, o_ref):
    o_ref[...] = _dot(x_ref[...], w_ref[...])


def _mm_asad(xp, w1sd):
    return pl.pallas_call(
        _k1b_body,
        grid=(RB,),
        in_specs=[
            pl.BlockSpec((256, FIN), lambda i: (i, 0)),
            pl.BlockSpec((FIN, 2 * HEADS), lambda i: (0, 0)),
        ],
        out_specs=pl.BlockSpec((256, 2 * HEADS), lambda i: (i, 0)),
        out_shape=jax.ShapeDtypeStruct((NP, 2 * HEADS), jnp.float32),
    )(xp, w1sd)


def _sc_pass(hh, as0, ad0, as1, ad1, dump, idx_v, eb_v, sc_v, al_v, rows_v,
             out_sh, d0_sh, d1_sh, sems, zb, zs, stripe, nh):
    pltpu.sync_copy(zb.at[stripe], out_sh.at[stripe])
    pltpu.sync_copy(zs.at[stripe], d0_sh.at[stripe])
    pltpu.sync_copy(zs.at[stripe], d1_sh.at[stripe])

    plsc.subcore_barrier()

    def issue(t, b):
        for g in range(K // L):
            gs = pl.ds(g * L, L)
            pk = idx_v[t, gs]
            eb_v[2 * b, gs] = pk & (PACK - 1)
            eb_v[2 * b + 1, gs] = lax.shift_right_logical(pk, 14)
        src_r = eb_v.at[2 * b]
        dst_r = eb_v.at[2 * b + 1]
        pltpu.async_copy(as0.at[src_r], sc_v.at[4 * b + 0], sems[b])
        pltpu.async_copy(ad0.at[dst_r], sc_v.at[4 * b + 1], sems[b])
        pltpu.async_copy(as1.at[src_r], sc_v.at[4 * b + 2], sems[b])
        pltpu.async_copy(ad1.at[dst_r], sc_v.at[4 * b + 3], sems[b])
        pltpu.async_copy(hh.at[src_r], rows_v.at[b], sems[b])

    def wait(b):
        pltpu.make_async_copy(as0.at[pl.ds(0, K)], sc_v.at[4 * b + 0], sems[b]).wait()
        pltpu.make_async_copy(ad0.at[pl.ds(0, K)], sc_v.at[4 * b + 1], sems[b]).wait()
        pltpu.make_async_copy(as1.at[pl.ds(0, K)], sc_v.at[4 * b + 2], sems[b]).wait()
        pltpu.make_async_copy(ad1.at[pl.ds(0, K)], sc_v.at[4 * b + 3], sems[b]).wait()
        pltpu.make_async_copy(hh.at[pl.ds(0, K)], rows_v.at[b], sems[b]).wait()

    def process(t, b):
        for g in range(K // L):
            gs = pl.ds(g * L, L)
            z0 = sc_v[4 * b + 0, gs] + sc_v[4 * b + 1, gs]
            z0 = jnp.where(z0 >= 0.0, z0, 0.2 * z0)
            al_v[2 * b, gs] = jnp.exp(z0)
            z1 = sc_v[4 * b + 2, gs] + sc_v[4 * b + 3, gs]
            z1 = jnp.where(z1 >= 0.0, z1, 0.2 * z1)
            al_v[2 * b + 1, gs] = jnp.exp(z1)
        def scale(e, c2):
            av0 = plsc.load_gather(al_v.at[2 * b], [jnp.full((L,), e, jnp.int32)])
            av1 = plsc.load_gather(al_v.at[2 * b + 1], [jnp.full((L,), e, jnp.int32)])
            for c in range(HID // L):
                rows_v[b, e, pl.ds(c * L, L)] = \
                    rows_v[b, e, pl.ds(c * L, L)] * av0
                rows_v[b, e, pl.ds(HID + c * L, L)] = \
                    rows_v[b, e, pl.ds(HID + c * L, L)] * av1
            return c2
        lax.fori_loop(0, K, scale, 0, unroll=4)
        for g in range(K // L):
            gs = pl.ds(g * L, L)
            eb_v[4, gs] = lax.shift_right_logical(idx_v[t, gs], 14)
        dsc = eb_v.at[4]
        pltpu.sync_copy(rows_v.at[b], out_sh.at[dsc], add=True)
        pltpu.sync_copy(al_v.at[2 * b], d0_sh.at[dsc], add=True)
        pltpu.sync_copy(al_v.at[2 * b + 1], d1_sh.at[dsc], add=True)

    issue(0, 0)
    issue(1, 1)

    def pair(tt, carry):
        t0 = 2 * tt
        wait(0)
        process(t0, 0)

        @pl.when(tt < nh - 1)
        def _():
            issue(t0 + 2, 0)

        wait(1)
        process(t0 + 1, 1)

        @pl.when(tt < nh - 1)
        def _():
            issue(t0 + 3, 1)

        return carry

    lax.fori_loop(0, nh, pair, 0)

    plsc.subcore_barrier()
    dump()
    plsc.subcore_barrier()


def _sc_body(h0, h1, h2, h3,
             t0, t1, t2, t3, t4, t5, t6, t7,
             t8, t9, t10, t11, t12, t13, t14, t15,
             epk, zb, zs, out, den,
             idx_v, eb_v, sc_v, al_v, rows_v,
             out_sh, d0_sh, d1_sh, semA, semB):
    cid = lax.axis_index("c")
    sid = lax.axis_index("s")
    wid = sid * NC + cid
    stripe = pl.ds(sid * STR, STR)
    sems = (semA, semB)
    hs = (h0, h1, h2, h3)
    ts = (t0, t1, t2, t3, t4, t5, t6, t7,
          t8, t9, t10, t11, t12, t13, t14, t15)
    nh = jnp.where(cid == 0, S0 // 2, S1 // 2)

    pltpu.sync_copy(epk.at[wid], idx_v)

    for p in range(HEADS // 2):
        def dump(p=p):
            pltpu.sync_copy(out_sh.at[stripe], out.at[p, cid, stripe])
            pltpu.sync_copy(d0_sh.at[stripe], den.at[p, cid, 0, stripe])
            pltpu.sync_copy(d1_sh.at[stripe], den.at[p, cid, 1, stripe])
        _sc_pass(hs[p], ts[2 * p], ts[HEADS + 2 * p],
                 ts[2 * p + 1], ts[HEADS + 2 * p + 1], dump,
                 idx_v, eb_v, sc_v, al_v, rows_v,
                 out_sh, d0_sh, d1_sh, sems, zb, zs, stripe, nh)


def _sc_body2(hh, as0, ad0, as1, ad1, epk, zb, zs, out, den,
              idx_v, eb_v, sc_v, al_v, rows_v,
              out_sh, d0_sh, d1_sh, semA, semB):
    cid = lax.axis_index("c")
    sid = lax.axis_index("s")
    wid = sid * NC + cid
    stripe = pl.ds(sid * STR, STR)

    pltpu.sync_copy(epk.at[wid], idx_v)
    nh = jnp.where(cid == 0, S0 // 2, S1 // 2)

    def dump():
        pltpu.sync_copy(out_sh.at[stripe], out.at[cid, stripe])
        pltpu.sync_copy(d0_sh.at[stripe], den.at[cid, 0, stripe])
        pltpu.sync_copy(d1_sh.at[stripe], den.at[cid, 1, stripe])

    _sc_pass(hh, as0, ad0, as1, ad1, dump,
             idx_v, eb_v, sc_v, al_v, rows_v,
             out_sh, d0_sh, d1_sh, (semA, semB), zb, zs, stripe, nh)


_SC_SCRATCH = [
    pltpu.VMEM((SMX, K), jnp.int32),
    pltpu.VMEM((8, K), jnp.int32),
    pltpu.VMEM((8, K), jnp.float32),
    pltpu.VMEM((4, K), jnp.float32),
    pltpu.VMEM((2, K, W), jnp.float32),
    pltpu.VMEM_SHARED((NP, W), jnp.float32),
    pltpu.VMEM_SHARED((NP,), jnp.float32),
    pltpu.VMEM_SHARED((NP,), jnp.float32),
    pltpu.SemaphoreType.DMA,
    pltpu.SemaphoreType.DMA,
]

_sc_gat1 = functools.partial(
    pl.kernel,
    out_type=(jax.ShapeDtypeStruct((HEADS // 2, NC, NP, W), jnp.float32),
              jax.ShapeDtypeStruct((HEADS // 2, NC, 2, NP), jnp.float32)),
    mesh=plsc.VectorSubcoreMesh(core_axis_name="c", subcore_axis_name="s",
                                num_cores=NC, num_subcores=NS),
    compiler_params=pltpu.CompilerParams(needs_layout_passes=False),
    scratch_types=_SC_SCRATCH,
)(_sc_body)

_sc_gat2 = functools.partial(
    pl.kernel,
    out_type=(jax.ShapeDtypeStruct((NC, NP, W), jnp.float32),
              jax.ShapeDtypeStruct((NC, 2, NP), jnp.float32)),
    mesh=plsc.VectorSubcoreMesh(core_axis_name="c", subcore_axis_name="s",
                                num_cores=NC, num_subcores=NS),
    compiler_params=pltpu.CompilerParams(needs_layout_passes=False),
    scratch_types=_SC_SCRATCH,
)(_sc_body2)


def _k3_body(o_ref, d_ref, b1_ref, w_ref, out_ref):
    i = pl.program_id(0)
    acc = jnp.zeros((256, 128), jnp.float32)
    for h in range(HEADS):
        p, q = h // 2, h % 2
        v = (o_ref[p, 0, :, q * HID:(q + 1) * HID]
             + o_ref[p, 1, :, q * HID:(q + 1) * HID])
        dh = d_ref[p, :, q, pl.ds(i * 256, 256)]
        dd = dh[0] + dh[1] + 1e-16
        v = v / dd[:, None] + b1_ref[h]
        v = jnp.where(v > 0.0, v, jnp.exp(v) - 1.0)
        acc = acc + _dot(v, w_ref[h])
    out_ref[...] = acc


def _combine_l2(out1, den1, b1r, w2cat):
    NPAIR = HEADS // 2
    return pl.pallas_call(
        _k3_body,
        grid=(RB,),
        in_specs=[
            pl.BlockSpec((NPAIR, NC, 256, W), lambda i: (0, 0, i, 0)),
            pl.BlockSpec((NPAIR, NC, 2, NP), lambda i: (0, 0, 0, 0)),
            pl.BlockSpec((HEADS, HID), lambda i: (0, 0)),
            pl.BlockSpec((HEADS, HID, 128), lambda i: (0, 0, 0)),
        ],
        out_specs=pl.BlockSpec((256, 128), lambda i: (i, 0)),
        out_shape=jax.ShapeDtypeStruct((NP, 128), jnp.float32),
    )(out1, den1, b1r, w2cat)


def _k4_body(o_ref, d_ref, b_ref, b2_ref, fcw_ref, fcb_ref, out_ref, sums, counts):
    i = pl.program_id(0)
    p = o_ref[0, :, :HID] + o_ref[1, :, :HID]
    dh = d_ref[:, 0, pl.ds(i * 256, 256)]
    dd = dh[0] + dh[1] + 1e-16
    v = p / dd[:, None] + b2_ref[0]
    v = jnp.where(v > 0.0, v, jnp.exp(v) - 1.0)
    bb = b_ref[0, 0]
    oh = (bb[:, None] == lax.broadcasted_iota(jnp.int32, (256, G), 1)).astype(jnp.float32)
    ps = jax.lax.dot_general(oh, v, (((0,), (0,)), ((), ())),
                             precision=_HI, preferred_element_type=jnp.float32)
    pc = jnp.sum(oh, axis=0)

    @pl.when(i == 0)
    def _():
        sums[...] = jnp.zeros_like(sums)
        counts[...] = jnp.zeros_like(counts)

    sums[...] += ps
    counts[...] += pc[None, :]

    @pl.when(i == RB - 1)
    def _():
        c = jnp.maximum(counts[0, :], 1.0)
        pooled = sums[...] / c[:, None]
        out_ref[...] = _dot(pooled, fcw_ref[...]) + fcb_ref[0]


def _pool_fc(out2, den2, batchr, b2r, fcw, fcb):
    return pl.pallas_call(
        _k4_body,
        grid=(RB,),
        in_specs=[
            pl.BlockSpec((NC, 256, W), lambda i: (0, i, 0)),
            pl.BlockSpec((NC, 2, NP), lambda i: (0, 0, 0)),
            pl.BlockSpec((1, 1, 256), lambda i: (i, 0, 0)),
            pl.BlockSpec((1, HID), lambda i: (0, 0)),
            pl.BlockSpec((HID, 128), lambda i: (0, 0)),
            pl.BlockSpec((1, 128), lambda i: (0, 0)),
        ],
        out_specs=pl.BlockSpec((G, 128), lambda i: (0, 0)),
        out_shape=jax.ShapeDtypeStruct((G, 128), jnp.float32),
        scratch_shapes=[pltpu.VMEM((G, HID), jnp.float32),
                        pltpu.VMEM((1, G), jnp.float32)],
    )(out2, den2, batchr, b2r, fcw, fcb)


def kernel(x, edge_index, batch, W1, att_src1, att_dst1, b1,
           W2, att_src2, att_dst2, b2, fc_w, fc_b):
    f32 = jnp.float32
    w1r = W1.reshape(FIN, HEADS // 2, W).transpose(1, 0, 2)
    w1s = jnp.einsum("fhc,hc->fh", W1.reshape(FIN, HEADS, HID), att_src1)
    w1d = jnp.einsum("fhc,hc->fh", W1.reshape(FIN, HEADS, HID), att_dst1)
    w1sd = jnp.concatenate([w1s, w1d], axis=1)
    w2r = W2.reshape(HEADS, HID, HID)
    w2s = (W2 @ att_src2[0]).reshape(HEADS, HID, 1)
    w2d = (W2 @ att_dst2[0]).reshape(HEADS, HID, 1)
    w2cat = jnp.concatenate(
        [w2r, w2s, w2d, jnp.zeros((HEADS, HID, 128 - HID - 2), f32)], axis=2)
    b1r = b1.reshape(HEADS, HID)
    b2r = b2.reshape(1, HID)
    fcw = jnp.zeros((HID, 128), f32).at[:, :2].set(fc_w)
    fcb = jnp.zeros((1, 128), f32).at[:, :2].set(fc_b)

    xp = jnp.pad(x, ((0, NP - N), (0, 0)))
    loop = jnp.arange(N, dtype=jnp.int32)
    pad = DUMMY + jnp.arange(ET_PAD - E_TOT, dtype=jnp.int32) % (NP - N)
    srcs = jnp.concatenate([edge_index[0], loop, pad])
    dsts = jnp.concatenate([edge_index[1], loop, pad])
    flat = dsts * PACK + srcs
    dfill = (DUMMY + jnp.arange(SMX * K, dtype=jnp.int32) % (NP - N)) * PACK \
        + DUMMY
    rows_list, off = [], 0
    for w in range(NW):
        lw = (S0 if w % NC == 0 else S1) * K
        seg = flat[off:off + lw]
        off += lw
        rows_list.append(jnp.concatenate([seg, dfill[:SMX * K - lw]]))
    epk = jnp.stack(rows_list).reshape(NW, SMX, K)
    batchr = jnp.concatenate(
        [batch, jnp.full((NP - N,), G, jnp.int32)]).reshape(RB, 1, 256)
    zb = jnp.zeros((NP, W), f32)
    zs = jnp.zeros((NP,), f32)

    h4 = _mm_heads(xp, w1r)
    asad = _mm_asad(xp, w1sd)
    asadt = asad.T

    hs = [h4[p] for p in range(HEADS // 2)]
    ts = [asadt[i] for i in range(2 * HEADS)]
    out1, den1 = _sc_gat1(*hs, *ts, epk, zb, zs)

    o2pre = _combine_l2(out1, den1, b1r, w2cat)
    as2 = o2pre[:, HID]
    ad2 = o2pre[:, HID + 1]

    out2, den2 = _sc_gat2(o2pre, as2, ad2, zs, zs, epk, zb, zs)

    logits = _pool_fc(out2, den2, batchr, b2r, fcw, fcb)
    return logits[:, :2]
